# baseline (speedup 1.0000x reference)
import functools

import jax
import jax.numpy as jnp
from jax.experimental import pallas as pl
from jax.experimental.pallas import tpu as pltpu

_B = 32
_NODES = 15
_INTERNAL = 8
_DRAFTS = 14
_WIDTH = 8
_BT = 2
_SUB = 8
_ROWLANES = 12500
_IMAX = jnp.iinfo(jnp.int32).max


def _tree_logic(idx, drafts):
    idx_f = idx.astype(jnp.float32)

    nn = jax.lax.broadcasted_iota(jnp.int32, (_INTERNAL, _DRAFTS), 0)
    dd = jax.lax.broadcasted_iota(jnp.int32, (_INTERNAL, _DRAFTS), 1)
    gather_parent = (nn == dd // 2).astype(jnp.float32)
    sampled = jax.lax.dot_general(
        idx_f, gather_parent, (((1,), (0,)), ((), ())),
        preferred_element_type=jnp.float32)
    acc = (sampled.astype(jnp.int32) == drafts).astype(jnp.float32)

    d14 = jax.lax.broadcasted_iota(jnp.int32, (_DRAFTS, _WIDTH), 0)
    w8 = jax.lax.broadcasted_iota(jnp.int32, (_DRAFTS, _WIDTH), 1)
    dot = functools.partial(jax.lax.dot_general,
                            dimension_numbers=(((1,), (0,)), ((), ())),
                            preferred_element_type=jnp.float32)
    ta0 = dot(acc, (d14 == w8 // 4).astype(jnp.float32))
    ta1 = dot(acc, (d14 == 2 + w8 // 2).astype(jnp.float32))
    ta2 = dot(acc, (d14 == 6 + w8).astype(jnp.float32))

    path_len = (ta0 + ta0 * ta1 + ta0 * ta1 * ta2).astype(jnp.int32)

    levels = jnp.max(path_len, axis=-1, keepdims=True)
    wi = jax.lax.broadcasted_iota(jnp.int32, (_BT, _WIDTH), 1)
    widx = jnp.min(jnp.where(path_len == levels, wi, _WIDTH),
                   axis=-1, keepdims=True)

    ap = jnp.where(levels == 0, 0,
                   jnp.where(levels == 1, 1 + widx // 4,
                             jnp.where(levels == 2, 3 + widx // 2,
                                       7 + widx)))

    a1 = ap + 1
    m1i = jax.lax.broadcasted_iota(jnp.int32, (_BT, _DRAFTS), 1) + 2
    depth_a = ((a1 >= 2).astype(jnp.int32) + (a1 >= 4).astype(jnp.int32)
               + (a1 >= 8).astype(jnp.int32))
    depth_m = ((m1i >= 2).astype(jnp.int32) + (m1i >= 4).astype(jnp.int32)
               + (m1i >= 8).astype(jnp.int32))
    shift = depth_a - depth_m
    anc = jnp.right_shift(a1, jnp.maximum(shift, 0)) == m1i
    mask = jnp.logical_and(shift >= 0, anc)
    return ap, mask


def _bonus_argmax(y):
    m = jnp.max(jnp.max(y, axis=2, keepdims=True), axis=1, keepdims=True)
    lane = jax.lax.broadcasted_iota(jnp.int32, y.shape, 2)
    sub = jax.lax.broadcasted_iota(jnp.int32, y.shape, 1)
    cand = jnp.where(y == m, sub * _ROWLANES + lane, _IMAX)
    idx = jnp.min(jnp.min(cand, axis=2, keepdims=True), axis=1, keepdims=True)
    return idx.reshape(_BT, 1)


def _fused_kernel(logits_ref, drafts_ref, hbm_ref,
                  out_tokens_ref, path_masks_ref, bonus_ref,
                  buf, rowsm, sems, *, gsteps):
    i = pl.program_id(0)
    slot = jax.lax.rem(i, 2)

    x = logits_ref[...]
    m = jnp.max(x, axis=-1, keepdims=True)
    cand = jnp.where(x == m,
                     jax.lax.broadcasted_iota(jnp.int32, x.shape, 2), _IMAX)
    idx = jnp.min(cand, axis=-1, keepdims=True).reshape(_BT, _INTERNAL)

    drafts = drafts_ref[pl.ds(i * _BT, _BT), :]
    ap, mask = _tree_logic(idx, drafts)

    out_tokens_ref[pl.ds(i * _BT, _BT), :] = jnp.where(mask, drafts, -1)
    path_masks_ref[pl.ds(i * _BT, _BT), :] = mask.astype(jnp.int32)

    for b in range(_BT):
        row = (i * _BT + b) * _NODES + ap[b, 0]
        rowsm[slot, b] = row
        pltpu.make_async_copy(hbm_ref.at[row], buf.at[slot, b],
                              sems.at[slot, b]).start()

    @pl.when(i > 0)
    def _drain_prev():
        pslot = 1 - slot
        for b in range(_BT):
            pltpu.make_async_copy(hbm_ref.at[rowsm[pslot, b]],
                                  buf.at[pslot, b], sems.at[pslot, b]).wait()
        bonus_ref[pl.ds((i - 1) * _BT, _BT), :] = _bonus_argmax(buf[pslot])

    @pl.when(i == gsteps - 1)
    def _drain_last():
        for b in range(_BT):
            pltpu.make_async_copy(hbm_ref.at[rowsm[slot, b]],
                                  buf.at[slot, b], sems.at[slot, b]).wait()
        bonus_ref[pl.ds(i * _BT, _BT), :] = _bonus_argmax(buf[slot])


def kernel(target_logits, draft_token_ids, tree_mask, tree_draft_positions):
    vocab = target_logits.shape[-1]
    gsteps = _B // _BT
    logits = target_logits[:_B * _NODES].reshape(_B, _NODES, vocab)
    hbm3 = target_logits[:_B * _NODES].reshape(_B * _NODES, _SUB, _ROWLANES)
    drafts = draft_token_ids.reshape(_B, _DRAFTS)

    out14, path_masks_i32, bonus = pl.pallas_call(
        functools.partial(_fused_kernel, gsteps=gsteps),
        grid=(gsteps,),
        in_specs=[
            pl.BlockSpec((_BT, _INTERNAL, vocab), lambda i: (i, 0, 0)),
            pl.BlockSpec((_B, _DRAFTS), lambda i: (0, 0)),
            pl.BlockSpec(memory_space=pltpu.MemorySpace.HBM),
        ],
        out_specs=[
            pl.BlockSpec((_B, _DRAFTS), lambda i: (0, 0)),
            pl.BlockSpec((_B, _DRAFTS), lambda i: (0, 0)),
            pl.BlockSpec((_B, 1), lambda i: (0, 0)),
        ],
        out_shape=[
            jax.ShapeDtypeStruct((_B, _DRAFTS), jnp.int32),
            jax.ShapeDtypeStruct((_B, _DRAFTS), jnp.int32),
            jax.ShapeDtypeStruct((_B, 1), jnp.int32),
        ],
        scratch_shapes=[
            pltpu.VMEM((2, _BT, _SUB, _ROWLANES), jnp.float32),
            pltpu.SMEM((2, _BT), jnp.int32),
            pltpu.SemaphoreType.DMA((2, _BT)),
        ],
    )(logits, drafts, hbm3)

    out_tokens = jnp.concatenate([out14, bonus], axis=1)
    return out_tokens, path_masks_i32.astype(jnp.bool_)


# speedup vs baseline: 1.0269x; 1.0269x over previous
import functools

import jax
import jax.numpy as jnp
from jax.experimental import pallas as pl
from jax.experimental.pallas import tpu as pltpu

_B = 32
_NODES = 15
_INTERNAL = 8
_DRAFTS = 14
_WIDTH = 8
_BT = 4
_SUB = 8
_ROWLANES = 12500
_IMAX = jnp.iinfo(jnp.int32).max


def _tree_logic(idx, drafts):
    idx_f = idx.astype(jnp.float32)

    nn = jax.lax.broadcasted_iota(jnp.int32, (_INTERNAL, _DRAFTS), 0)
    dd = jax.lax.broadcasted_iota(jnp.int32, (_INTERNAL, _DRAFTS), 1)
    gather_parent = (nn == dd // 2).astype(jnp.float32)
    sampled = jax.lax.dot_general(
        idx_f, gather_parent, (((1,), (0,)), ((), ())),
        preferred_element_type=jnp.float32)
    acc = (sampled.astype(jnp.int32) == drafts).astype(jnp.float32)

    d14 = jax.lax.broadcasted_iota(jnp.int32, (_DRAFTS, _WIDTH), 0)
    w8 = jax.lax.broadcasted_iota(jnp.int32, (_DRAFTS, _WIDTH), 1)
    dot = functools.partial(jax.lax.dot_general,
                            dimension_numbers=(((1,), (0,)), ((), ())),
                            preferred_element_type=jnp.float32)
    ta0 = dot(acc, (d14 == w8 // 4).astype(jnp.float32))
    ta1 = dot(acc, (d14 == 2 + w8 // 2).astype(jnp.float32))
    ta2 = dot(acc, (d14 == 6 + w8).astype(jnp.float32))

    path_len = (ta0 + ta0 * ta1 + ta0 * ta1 * ta2).astype(jnp.int32)

    levels = jnp.max(path_len, axis=-1, keepdims=True)
    wi = jax.lax.broadcasted_iota(jnp.int32, (_BT, _WIDTH), 1)
    widx = jnp.min(jnp.where(path_len == levels, wi, _WIDTH),
                   axis=-1, keepdims=True)

    ap = jnp.where(levels == 0, 0,
                   jnp.where(levels == 1, 1 + widx // 4,
                             jnp.where(levels == 2, 3 + widx // 2,
                                       7 + widx)))

    a1 = ap + 1
    m1i = jax.lax.broadcasted_iota(jnp.int32, (_BT, _DRAFTS), 1) + 2
    depth_a = ((a1 >= 2).astype(jnp.int32) + (a1 >= 4).astype(jnp.int32)
               + (a1 >= 8).astype(jnp.int32))
    depth_m = ((m1i >= 2).astype(jnp.int32) + (m1i >= 4).astype(jnp.int32)
               + (m1i >= 8).astype(jnp.int32))
    shift = depth_a - depth_m
    anc = jnp.right_shift(a1, jnp.maximum(shift, 0)) == m1i
    mask = jnp.logical_and(shift >= 0, anc)
    return ap, mask


def _bonus_argmax(y):
    m = jnp.max(jnp.max(y, axis=2, keepdims=True), axis=1, keepdims=True)
    lane = jax.lax.broadcasted_iota(jnp.int32, y.shape, 2)
    sub = jax.lax.broadcasted_iota(jnp.int32, y.shape, 1)
    cand = jnp.where(y == m, sub * _ROWLANES + lane, _IMAX)
    idx = jnp.min(jnp.min(cand, axis=2, keepdims=True), axis=1, keepdims=True)
    return idx.reshape(_BT, 1)


def _fused_kernel(logits_ref, drafts_ref, hbm_ref,
                  out_tokens_ref, path_masks_ref, bonus_ref,
                  buf, rowsm, sems, *, gsteps):
    i = pl.program_id(0)
    slot = jax.lax.rem(i, 2)

    x = logits_ref[...]
    m = jnp.max(x, axis=-1, keepdims=True)
    cand = jnp.where(x == m,
                     jax.lax.broadcasted_iota(jnp.int32, x.shape, 2), _IMAX)
    idx = jnp.min(cand, axis=-1, keepdims=True).reshape(_BT, _INTERNAL)

    drafts = drafts_ref[pl.ds(i * _BT, _BT), :]
    ap, mask = _tree_logic(idx, drafts)

    out_tokens_ref[pl.ds(i * _BT, _BT), :] = jnp.where(mask, drafts, -1)
    path_masks_ref[pl.ds(i * _BT, _BT), :] = mask.astype(jnp.int32)

    for b in range(_BT):
        row = (i * _BT + b) * _NODES + ap[b, 0]
        rowsm[slot, b] = row
        pltpu.make_async_copy(hbm_ref.at[row], buf.at[slot, b],
                              sems.at[slot, b]).start()

    @pl.when(i > 0)
    def _drain_prev():
        pslot = 1 - slot
        for b in range(_BT):
            pltpu.make_async_copy(hbm_ref.at[rowsm[pslot, b]],
                                  buf.at[pslot, b], sems.at[pslot, b]).wait()
        bonus_ref[pl.ds((i - 1) * _BT, _BT), :] = _bonus_argmax(buf[pslot])

    @pl.when(i == gsteps - 1)
    def _drain_last():
        for b in range(_BT):
            pltpu.make_async_copy(hbm_ref.at[rowsm[slot, b]],
                                  buf.at[slot, b], sems.at[slot, b]).wait()
        bonus_ref[pl.ds(i * _BT, _BT), :] = _bonus_argmax(buf[slot])


def kernel(target_logits, draft_token_ids, tree_mask, tree_draft_positions):
    vocab = target_logits.shape[-1]
    gsteps = _B // _BT
    logits = target_logits[:_B * _NODES].reshape(_B, _NODES, vocab)
    hbm3 = target_logits[:_B * _NODES].reshape(_B * _NODES, _SUB, _ROWLANES)
    drafts = draft_token_ids.reshape(_B, _DRAFTS)

    out14, path_masks_i32, bonus = pl.pallas_call(
        functools.partial(_fused_kernel, gsteps=gsteps),
        grid=(gsteps,),
        in_specs=[
            pl.BlockSpec((_BT, _INTERNAL, vocab), lambda i: (i, 0, 0)),
            pl.BlockSpec((_B, _DRAFTS), lambda i: (0, 0)),
            pl.BlockSpec(memory_space=pltpu.MemorySpace.HBM),
        ],
        out_specs=[
            pl.BlockSpec((_B, _DRAFTS), lambda i: (0, 0)),
            pl.BlockSpec((_B, _DRAFTS), lambda i: (0, 0)),
            pl.BlockSpec((_B, 1), lambda i: (0, 0)),
        ],
        out_shape=[
            jax.ShapeDtypeStruct((_B, _DRAFTS), jnp.int32),
            jax.ShapeDtypeStruct((_B, _DRAFTS), jnp.int32),
            jax.ShapeDtypeStruct((_B, 1), jnp.int32),
        ],
        scratch_shapes=[
            pltpu.VMEM((2, _BT, _SUB, _ROWLANES), jnp.float32),
            pltpu.SMEM((2, _BT), jnp.int32),
            pltpu.SemaphoreType.DMA((2, _BT)),
        ],
    )(logits, drafts, hbm3)

    out_tokens = jnp.concatenate([out14, bonus], axis=1)
    return out_tokens, path_masks_i32.astype(jnp.bool_)
